# Initial kernel scaffold; baseline (speedup 1.0000x reference)
import jax
import jax.numpy as jnp
from jax.experimental import pallas as pl


def kernel(x, sel, w):
    raise NotImplementedError("write your pallas kernel here")


# speedup vs baseline: 11.9298x; 11.9298x over previous
import functools

import jax
import jax.numpy as jnp
from jax.experimental import pallas as pl
from jax.experimental.pallas import tpu as pltpu

BM = 512


def _masked_body(x_ref, sel_ref, w_ref, o_ref):
    e = pl.program_id(1)
    p = jnp.dot(x_ref[...], w_ref[0], preferred_element_type=jnp.float32)
    mask = sel_ref[...] == e
    o_ref[...] = jnp.where(mask, p, o_ref[...])


def kernel(x, sel, w):
    M, K = x.shape
    E, _, N = w.shape
    sel2 = sel.reshape(M, 1)
    grid = (M // BM, E)
    out = pl.pallas_call(
        _masked_body,
        grid=grid,
        in_specs=[
            pl.BlockSpec((BM, K), lambda m, e: (m, 0)),
            pl.BlockSpec((BM, 1), lambda m, e: (m, 0)),
            pl.BlockSpec((1, K, N), lambda m, e: (e, 0, 0)),
        ],
        out_specs=pl.BlockSpec((BM, N), lambda m, e: (m, 0)),
        out_shape=jax.ShapeDtypeStruct((M, N), jnp.float32),
        compiler_params=pltpu.CompilerParams(
            dimension_semantics=("parallel", "arbitrary"),
        ),
    )(x, sel2, w)
    return out


# speedup vs baseline: 12.0038x; 1.0062x over previous
import functools

import jax
import jax.numpy as jnp
from jax.experimental import pallas as pl
from jax.experimental.pallas import tpu as pltpu

BM = 512


def _masked_body(x_ref, sel_ref, w_ref, o_ref):
    e = pl.program_id(1)
    p = jnp.dot(x_ref[...], w_ref[0], preferred_element_type=jnp.float32)
    mask = sel_ref[...] == e
    o_ref[...] = jnp.where(mask, p, o_ref[...])


def kernel(x, sel, w):
    M, K = x.shape
    E, _, N = w.shape
    x = x.astype(jnp.bfloat16)
    w = w.astype(jnp.bfloat16)
    sel2 = sel.reshape(M, 1)
    grid = (M // BM, E)
    out = pl.pallas_call(
        _masked_body,
        grid=grid,
        in_specs=[
            pl.BlockSpec((BM, K), lambda m, e: (m, 0)),
            pl.BlockSpec((BM, 1), lambda m, e: (m, 0)),
            pl.BlockSpec((1, K, N), lambda m, e: (e, 0, 0)),
        ],
        out_specs=pl.BlockSpec((BM, N), lambda m, e: (m, 0)),
        out_shape=jax.ShapeDtypeStruct((M, N), jnp.float32),
        compiler_params=pltpu.CompilerParams(
            dimension_semantics=("parallel", "arbitrary"),
        ),
    )(x, sel2, w)
    return out


# speedup vs baseline: 24.5311x; 2.0436x over previous
import functools

import jax
import jax.numpy as jnp
from jax import lax
from jax.experimental import pallas as pl
from jax.experimental.pallas import tpu as pltpu
from jax.experimental.pallas import tpu_sc as plsc

BM = 256


def _sc_permute_rows(src, idx, invert):
    M, D = src.shape
    info = plsc.get_sparse_core_info()
    NC, NS = info.num_cores, info.num_subcores
    NW = NC * NS
    per_w = M // NW
    ch = per_w
    while ch * D * 4 > 256 * 1024:
        ch //= 2
    mesh = plsc.VectorSubcoreMesh(core_axis_name="c", subcore_axis_name="s")

    @functools.partial(
        pl.kernel,
        out_type=jax.ShapeDtypeStruct((M, D), src.dtype),
        mesh=mesh,
        scratch_types=[
            pltpu.VMEM((ch,), jnp.int32),
            pltpu.VMEM((ch, D), src.dtype),
            pltpu.SemaphoreType.DMA,
        ],
    )
    def body(src_hbm, idx_hbm, out_hbm, idx_v, rows_v, sem):
        wid = lax.axis_index("s") * NC + lax.axis_index("c")
        for c in range(per_w // ch):
            base = wid * per_w + c * ch
            pltpu.sync_copy(idx_hbm.at[pl.ds(base, ch)], idx_v)
            if invert:
                pltpu.async_copy(src_hbm.at[idx_v], rows_v, sem).wait()
                pltpu.sync_copy(rows_v, out_hbm.at[pl.ds(base, ch)])
            else:
                pltpu.sync_copy(src_hbm.at[pl.ds(base, ch)], rows_v)
                pltpu.async_copy(rows_v, out_hbm.at[idx_v], sem).wait()

    return body(src, idx)


def _gmm_body(grp_ref, tile_ref, lo_ref, hi_ref, x_ref, w_ref, o_ref):
    i = pl.program_id(0)
    rows = lax.broadcasted_iota(jnp.int32, (BM, 1), 0)
    mask = (rows >= lo_ref[i]) & (rows < hi_ref[i])
    p = jnp.dot(x_ref[...], w_ref[0], preferred_element_type=jnp.float32)
    o_ref[...] = jnp.where(mask, p, o_ref[...])


def _tc_gmm(x_sorted, w, grp, tile, lo, hi, steps):
    M, K = x_sorted.shape
    E, _, N = w.shape
    grid_spec = pltpu.PrefetchScalarGridSpec(
        num_scalar_prefetch=4,
        grid=(steps,),
        in_specs=[
            pl.BlockSpec((BM, K), lambda i, grp, tile, lo, hi: (tile[i], 0)),
            pl.BlockSpec((1, K, N), lambda i, grp, tile, lo, hi: (grp[i], 0, 0)),
        ],
        out_specs=pl.BlockSpec((BM, N), lambda i, grp, tile, lo, hi: (tile[i], 0)),
    )
    return pl.pallas_call(
        _gmm_body,
        grid_spec=grid_spec,
        out_shape=jax.ShapeDtypeStruct((M, N), jnp.float32),
        compiler_params=pltpu.CompilerParams(
            dimension_semantics=("arbitrary",),
        ),
    )(grp, tile, lo, hi, x_sorted, w)


def _routing_metadata(sel, M, E):
    i32 = jnp.int32
    oh = (sel[:, None] == jnp.arange(E, dtype=sel.dtype)[None, :]).astype(i32)
    cum = jnp.cumsum(oh, axis=0)
    cnt = cum[-1]
    starts = jnp.concatenate([jnp.zeros(1, i32), jnp.cumsum(cnt)[:-1].astype(i32)])
    ends = starts + cnt
    rank = jnp.take_along_axis(cum, sel[:, None].astype(i32), 1)[:, 0] - 1
    pos = starts[sel] + rank

    T = M // BM
    S = T + E - 1
    first_tile = starts // BM
    ntiles = jnp.where(cnt > 0, (ends + BM - 1) // BM - first_tile, 0)
    incl = jnp.cumsum(ntiles)
    total = incl[-1]
    step_start = incl - ntiles
    i = jnp.arange(S, dtype=i32)
    g = jnp.minimum(jnp.sum(incl[None, :] <= i[:, None], axis=1), E - 1)
    tile = first_tile[g] + (i - step_start[g])
    valid = i < total
    tile = jnp.where(valid, tile, T - 1).astype(i32)
    lo = jnp.where(valid, jnp.clip(starts[g] - tile * BM, 0, BM), 0).astype(i32)
    hi = jnp.where(valid, jnp.clip(ends[g] - tile * BM, 0, BM), 0).astype(i32)
    grp = jnp.where(valid, g, E - 1).astype(i32)
    return pos.astype(i32), grp, tile, lo, hi, S


def kernel(x, sel, w):
    M, K = x.shape
    E, _, N = w.shape
    pos, grp, tile, lo, hi, steps = _routing_metadata(sel, M, E)
    x_sorted = _sc_permute_rows(x, pos, invert=False)
    out_sorted = _tc_gmm(x_sorted, w, grp, tile, lo, hi, steps)
    return _sc_permute_rows(out_sorted, pos, invert=True)


# speedup vs baseline: 29.7883x; 1.2143x over previous
import functools

import jax
import jax.numpy as jnp
from jax import lax
from jax.experimental import pallas as pl
from jax.experimental.pallas import tpu as pltpu
from jax.experimental.pallas import tpu_sc as plsc

BM = 256


def _sc_permute_rows(src, idx, invert):
    M, D = src.shape
    info = plsc.get_sparse_core_info()
    NC, NS = info.num_cores, info.num_subcores
    NW = NC * NS
    per_w = M // NW
    ch = per_w
    while ch * D * 4 > 256 * 1024:
        ch //= 2
    mesh = plsc.VectorSubcoreMesh(core_axis_name="c", subcore_axis_name="s")

    @functools.partial(
        pl.kernel,
        out_type=jax.ShapeDtypeStruct((M, D), src.dtype),
        mesh=mesh,
        scratch_types=[
            pltpu.VMEM((ch,), jnp.int32),
            pltpu.VMEM((ch, D), src.dtype),
            pltpu.SemaphoreType.DMA,
        ],
    )
    def body(src_hbm, idx_hbm, out_hbm, idx_v, rows_v, sem):
        wid = lax.axis_index("s") * NC + lax.axis_index("c")
        for c in range(per_w // ch):
            base = wid * per_w + c * ch
            pltpu.sync_copy(idx_hbm.at[pl.ds(base, ch)], idx_v)
            if invert:
                pltpu.async_copy(src_hbm.at[idx_v], rows_v, sem).wait()
                pltpu.sync_copy(rows_v, out_hbm.at[pl.ds(base, ch)])
            else:
                pltpu.sync_copy(src_hbm.at[pl.ds(base, ch)], rows_v)
                pltpu.async_copy(rows_v, out_hbm.at[idx_v], sem).wait()

    return body(src, idx)


def _gmm_body(grp_ref, tile_ref, lo_ref, hi_ref, x_ref, w_ref, o_ref):
    i = pl.program_id(0)
    rows = lax.broadcasted_iota(jnp.int32, (BM, 1), 0)
    mask = (rows >= lo_ref[i]) & (rows < hi_ref[i])
    p = jnp.dot(x_ref[...], w_ref[0], preferred_element_type=jnp.float32)
    o_ref[...] = jnp.where(mask, p, o_ref[...])


def _tc_gmm(x_sorted, w, grp, tile, lo, hi, steps):
    M, K = x_sorted.shape
    E, _, N = w.shape
    grid_spec = pltpu.PrefetchScalarGridSpec(
        num_scalar_prefetch=4,
        grid=(steps,),
        in_specs=[
            pl.BlockSpec((BM, K), lambda i, grp, tile, lo, hi: (tile[i], 0)),
            pl.BlockSpec((1, K, N), lambda i, grp, tile, lo, hi: (grp[i], 0, 0)),
        ],
        out_specs=pl.BlockSpec((BM, N), lambda i, grp, tile, lo, hi: (tile[i], 0)),
    )
    return pl.pallas_call(
        _gmm_body,
        grid_spec=grid_spec,
        out_shape=jax.ShapeDtypeStruct((M, N), jnp.float32),
        compiler_params=pltpu.CompilerParams(
            dimension_semantics=("arbitrary",),
        ),
    )(grp, tile, lo, hi, x_sorted, w)


def _routing_metadata(sel, M, E):
    i32 = jnp.int32
    B = 512
    G = M // B
    oh = (sel[:, None] == jnp.arange(E, dtype=sel.dtype)[None, :]).astype(jnp.float32)
    ohb = oh.reshape(G, B, E)
    tri = jnp.tril(jnp.ones((B, B), jnp.float32))
    within = jax.lax.dot_general(tri, ohb, (((1,), (1,)), ((), ())))
    within = within.transpose(1, 0, 2)
    blocksum = within[:, -1, :]
    blockpref = jnp.cumsum(blocksum, axis=0) - blocksum
    cumf = (within + blockpref[:, None, :]).reshape(M, E)
    cnt = (blocksum[-1] + blockpref[-1]).astype(i32)
    starts = jnp.concatenate([jnp.zeros(1, i32), jnp.cumsum(cnt)[:-1].astype(i32)])
    ends = starts + cnt
    rank = jnp.sum(cumf * oh, axis=1).astype(i32) - 1
    pos = jnp.sum(starts[None, :].astype(jnp.float32) * oh, axis=1).astype(i32) + rank

    T = M // BM
    S = T + E - 1
    first_tile = starts // BM
    ntiles = jnp.where(cnt > 0, (ends + BM - 1) // BM - first_tile, 0)
    incl = jnp.cumsum(ntiles)
    total = incl[-1]
    step_start = incl - ntiles
    i = jnp.arange(S, dtype=i32)
    g = jnp.minimum(jnp.sum(incl[None, :] <= i[:, None], axis=1), E - 1)
    tile = first_tile[g] + (i - step_start[g])
    valid = i < total
    tile = jnp.where(valid, tile, T - 1).astype(i32)
    lo = jnp.where(valid, jnp.clip(starts[g] - tile * BM, 0, BM), 0).astype(i32)
    hi = jnp.where(valid, jnp.clip(ends[g] - tile * BM, 0, BM), 0).astype(i32)
    grp = jnp.where(valid, g, E - 1).astype(i32)
    return pos.astype(i32), grp, tile, lo, hi, S


def kernel(x, sel, w):
    M, K = x.shape
    E, _, N = w.shape
    pos, grp, tile, lo, hi, steps = _routing_metadata(sel, M, E)
    x_sorted = _sc_permute_rows(x, pos, invert=False)
    out_sorted = _tc_gmm(x_sorted, w, grp, tile, lo, hi, steps)
    return _sc_permute_rows(out_sorted, pos, invert=True)


# speedup vs baseline: 30.1633x; 1.0126x over previous
import functools

import jax
import jax.numpy as jnp
from jax import lax
from jax.experimental import pallas as pl
from jax.experimental.pallas import tpu as pltpu
from jax.experimental.pallas import tpu_sc as plsc

BM = 256


def _sc_scatter_rows(src, idx):
    M, D = src.shape
    info = plsc.get_sparse_core_info()
    NC, NS = info.num_cores, info.num_subcores
    NW = NC * NS
    per_w = M // NW
    ch = per_w
    while 2 * ch * D * 4 > 500 * 1024:
        ch //= 2
    nch = per_w // ch
    idx2 = idx.reshape(M // ch, ch)
    mesh = plsc.VectorSubcoreMesh(core_axis_name="c", subcore_axis_name="s")

    @functools.partial(
        pl.kernel,
        out_type=jax.ShapeDtypeStruct((M, D), src.dtype),
        mesh=mesh,
        scratch_types=[
            pltpu.VMEM((nch, ch), jnp.int32),
            pltpu.VMEM((ch, D), src.dtype),
            pltpu.VMEM((ch, D), src.dtype),
            pltpu.SemaphoreType.DMA,
            pltpu.SemaphoreType.DMA,
            pltpu.SemaphoreType.DMA,
            pltpu.SemaphoreType.DMA,
        ],
    )
    def body(src_hbm, idx_hbm, out_hbm, idx_v, row0, row1, ls0, ls1, ss0, ss1):
        wid = lax.axis_index("s") * NC + lax.axis_index("c")
        base = wid * per_w
        pltpu.sync_copy(idx_hbm.at[pl.ds(wid * nch, nch)], idx_v)
        rows = (row0, row1)
        lsem = (ls0, ls1)
        ssem = (ss0, ss1)
        loads = [None, None]
        scats = [None, None]
        loads[0] = pltpu.async_copy(src_hbm.at[pl.ds(base, ch)], rows[0], lsem[0])
        for c in range(nch):
            b = c & 1
            nb = 1 - b
            if c + 1 < nch:
                if scats[nb] is not None:
                    scats[nb].wait()
                loads[nb] = pltpu.async_copy(
                    src_hbm.at[pl.ds(base + (c + 1) * ch, ch)], rows[nb], lsem[nb])
            loads[b].wait()
            scats[b] = pltpu.async_copy(rows[b], out_hbm.at[idx_v.at[c]], ssem[b])
        for s in scats:
            if s is not None:
                s.wait()

    return body(src, idx2)


def _sc_gather_rows(src, idx):
    M = idx.shape[0]
    D = src.shape[1]
    info = plsc.get_sparse_core_info()
    NC, NS = info.num_cores, info.num_subcores
    NW = NC * NS
    per_w = M // NW
    ch = per_w
    while ch * D * 4 > 256 * 1024:
        ch //= 2
    mesh = plsc.VectorSubcoreMesh(core_axis_name="c", subcore_axis_name="s")

    @functools.partial(
        pl.kernel,
        out_type=jax.ShapeDtypeStruct((M, D), src.dtype),
        mesh=mesh,
        scratch_types=[
            pltpu.VMEM((ch,), jnp.int32),
            pltpu.VMEM((ch, D), src.dtype),
            pltpu.SemaphoreType.DMA,
        ],
    )
    def body(src_hbm, idx_hbm, out_hbm, idx_v, rows_v, sem):
        wid = lax.axis_index("s") * NC + lax.axis_index("c")
        for c in range(per_w // ch):
            base = wid * per_w + c * ch
            pltpu.sync_copy(idx_hbm.at[pl.ds(base, ch)], idx_v)
            pltpu.async_copy(src_hbm.at[idx_v], rows_v, sem).wait()
            pltpu.sync_copy(rows_v, out_hbm.at[pl.ds(base, ch)])

    return body(src, idx)


def _gmm_body(grp_ref, tile_ref, lo_ref, hi_ref, x_ref, w_ref, o_ref):
    i = pl.program_id(0)
    rows = lax.broadcasted_iota(jnp.int32, (BM, 1), 0)
    mask = (rows >= lo_ref[i]) & (rows < hi_ref[i])
    p = jnp.dot(x_ref[...], w_ref[0], preferred_element_type=jnp.float32)
    o_ref[...] = jnp.where(mask, p, o_ref[...])


def _tc_gmm(x_sorted, w, grp, tile, lo, hi, steps):
    M, K = x_sorted.shape
    E, _, N = w.shape
    grid_spec = pltpu.PrefetchScalarGridSpec(
        num_scalar_prefetch=4,
        grid=(steps,),
        in_specs=[
            pl.BlockSpec((BM, K), lambda i, grp, tile, lo, hi: (tile[i], 0)),
            pl.BlockSpec((1, K, N), lambda i, grp, tile, lo, hi: (grp[i], 0, 0)),
        ],
        out_specs=pl.BlockSpec((BM, N), lambda i, grp, tile, lo, hi: (tile[i], 0)),
    )
    return pl.pallas_call(
        _gmm_body,
        grid_spec=grid_spec,
        out_shape=jax.ShapeDtypeStruct((M, N), jnp.float32),
        compiler_params=pltpu.CompilerParams(
            dimension_semantics=("arbitrary",),
        ),
    )(grp, tile, lo, hi, x_sorted, w)


def _routing_metadata(sel, M, E):
    i32 = jnp.int32
    B = 512
    G = M // B
    oh = (sel[:, None] == jnp.arange(E, dtype=sel.dtype)[None, :]).astype(jnp.float32)
    ohb = oh.reshape(G, B, E)
    tri = jnp.tril(jnp.ones((B, B), jnp.float32))
    within = jax.lax.dot_general(tri, ohb, (((1,), (1,)), ((), ())))
    within = within.transpose(1, 0, 2)
    blocksum = within[:, -1, :]
    blockpref = jnp.cumsum(blocksum, axis=0) - blocksum
    cumf = (within + blockpref[:, None, :]).reshape(M, E)
    cnt = (blocksum[-1] + blockpref[-1]).astype(i32)
    starts = jnp.concatenate([jnp.zeros(1, i32), jnp.cumsum(cnt)[:-1].astype(i32)])
    ends = starts + cnt
    rank = jnp.sum(cumf * oh, axis=1).astype(i32) - 1
    pos = jnp.sum(starts[None, :].astype(jnp.float32) * oh, axis=1).astype(i32) + rank

    T = M // BM
    S = T + E - 1
    first_tile = starts // BM
    ntiles = jnp.where(cnt > 0, (ends + BM - 1) // BM - first_tile, 0)
    incl = jnp.cumsum(ntiles)
    total = incl[-1]
    step_start = incl - ntiles
    i = jnp.arange(S, dtype=i32)
    g = jnp.minimum(jnp.sum(incl[None, :] <= i[:, None], axis=1), E - 1)
    tile = first_tile[g] + (i - step_start[g])
    valid = i < total
    tile = jnp.where(valid, tile, T - 1).astype(i32)
    lo = jnp.where(valid, jnp.clip(starts[g] - tile * BM, 0, BM), 0).astype(i32)
    hi = jnp.where(valid, jnp.clip(ends[g] - tile * BM, 0, BM), 0).astype(i32)
    grp = jnp.where(valid, g, E - 1).astype(i32)
    return pos.astype(i32), grp, tile, lo, hi, S


def kernel(x, sel, w):
    M, K = x.shape
    E, _, N = w.shape
    pos, grp, tile, lo, hi, steps = _routing_metadata(sel, M, E)
    x_sorted = _sc_scatter_rows(x, pos)
    out_sorted = _tc_gmm(x_sorted, w, grp, tile, lo, hi, steps)
    return _sc_gather_rows(out_sorted, pos)
